# baseline (speedup 1.0000x reference)
import functools

import jax
import jax.numpy as jnp
from jax import lax
from jax.experimental import pallas as pl
from jax.experimental.pallas import tpu as pltpu
from jax.experimental.pallas import tpu_sc as plsc

NC = 2
NS = 16
NW = NC * NS

D = 64
CH = 128


NBUF = 4
CH = 128
DP = 128


def _make_gather(total):
    rows_w = total // NW
    nch = rows_w // CH
    ngrp = nch // NBUF
    assert total % (NW * CH * NBUF) == 0
    mesh = plsc.VectorSubcoreMesh(core_axis_name="c", subcore_axis_name="s")

    @functools.partial(
        pl.kernel,
        mesh=mesh,
        compiler_params=pltpu.CompilerParams(use_tc_tiling_on_sc=True),
        out_type=jax.ShapeDtypeStruct((total, DP), jnp.float32),
        scratch_types=[
            pltpu.VMEM((rows_w,), jnp.int32),
            pltpu.VMEM((NBUF, CH, DP), jnp.float32),
            pltpu.SemaphoreType.DMA((NBUF,)),
        ],
    )
    def gather(idx_hbm, table_hbm, out_hbm, idx_v, rows_v, sem):
        wid = lax.axis_index("s") * NC + lax.axis_index("c")
        base = wid * rows_w
        pltpu.sync_copy(idx_hbm.at[pl.ds(base, rows_w)], idx_v)

        def start(c, b):
            pltpu.async_copy(
                table_hbm.at[idx_v.at[pl.ds(c * CH, CH)]], rows_v.at[b], sem.at[b]
            )

        def finish(c, b):
            pltpu.make_async_copy(
                table_hbm.at[idx_v.at[pl.ds(c * CH, CH)]], rows_v.at[b], sem.at[b]
            ).wait()
            pltpu.sync_copy(rows_v.at[b], out_hbm.at[pl.ds(base + c * CH, CH)])

        for b in range(NBUF):
            start(b, b)

        def group(g, carry):
            for b in range(NBUF):
                c = g * NBUF + b
                finish(c, b)
                start(c + NBUF, b)
            return carry

        lax.fori_loop(0, ngrp - 1, group, 0)

        for b in range(NBUF):
            finish((ngrp - 1) * NBUF + b, b)

    return gather


def kernel(indices, weight):
    batch, hist = indices.shape
    var_len, d = weight.shape
    wpad = jnp.pad(weight, ((0, 0), (0, DP - d)))
    idxf = indices.reshape(batch * hist)
    out2 = _make_gather(batch * hist)(idxf, wpad)
    return out2[:, :d].reshape(batch, hist, d)


# speedup vs baseline: 1.1377x; 1.1377x over previous
import functools

import jax
import jax.numpy as jnp
from jax import lax
from jax.experimental import pallas as pl
from jax.experimental.pallas import tpu as pltpu
from jax.experimental.pallas import tpu_sc as plsc

NC = 2
NS = 16
NW = NC * NS

D = 64
CH = 128
NBUF = 8


def _make_gather(batch):
    assert batch % (NW * CH * NBUF) == 0
    nch = batch // (NW * CH)
    ngrp = nch // NBUF
    mesh = plsc.VectorSubcoreMesh(core_axis_name="c", subcore_axis_name="s")

    @functools.partial(
        pl.kernel,
        mesh=mesh,
        compiler_params=pltpu.CompilerParams(use_tc_tiling_on_sc=False),
        out_type=jax.ShapeDtypeStruct((NW, nch, CH, D), jnp.float32),
        scratch_types=[
            pltpu.VMEM((nch, CH), jnp.int32),
            pltpu.VMEM((NBUF, CH, D), jnp.float32),
            pltpu.SemaphoreType.DMA((NBUF,)),
        ],
    )
    def gather(idx_hbm, table_hbm, out_hbm, idx_v, rows_v, sem):
        wid = lax.axis_index("s") * NC + lax.axis_index("c")
        pltpu.sync_copy(idx_hbm.at[wid], idx_v)

        for b in range(NBUF):
            pltpu.async_copy(table_hbm.at[idx_v.at[b]], rows_v.at[b], sem.at[b])

        def group(g, carry):
            for b in range(NBUF):
                c = g * NBUF + b
                pltpu.make_async_copy(
                    table_hbm.at[idx_v.at[c]], rows_v.at[b], sem.at[b]
                ).wait()
                pltpu.sync_copy(rows_v.at[b], out_hbm.at[wid, c])
                pltpu.async_copy(
                    table_hbm.at[idx_v.at[c + NBUF]], rows_v.at[b], sem.at[b]
                )
            return carry

        lax.fori_loop(0, ngrp - 1, group, 0)

        for b in range(NBUF):
            c = (ngrp - 1) * NBUF + b
            pltpu.make_async_copy(
                table_hbm.at[idx_v.at[c]], rows_v.at[b], sem.at[b]
            ).wait()
            pltpu.sync_copy(rows_v.at[b], out_hbm.at[wid, c])

    return gather


def kernel(indices, weight):
    batch, hist = indices.shape
    total = batch * hist
    idx = indices.reshape(NW, total // (NW * CH), CH).astype(jnp.int32)
    out = _make_gather(total)(idx, weight)
    return out.reshape(batch, hist, D)


# speedup vs baseline: 1.1398x; 1.0019x over previous
import functools

import jax
import jax.numpy as jnp
from jax import lax
from jax.experimental import pallas as pl
from jax.experimental.pallas import tpu as pltpu
from jax.experimental.pallas import tpu_sc as plsc

NC = 2
NS = 16
NW = NC * NS

D = 64
CH = 128
NBUF = 4


def _make_gather(batch):
    assert batch % (NW * CH * NBUF) == 0
    nch = batch // (NW * CH)
    ngrp = nch // NBUF
    mesh = plsc.VectorSubcoreMesh(core_axis_name="c", subcore_axis_name="s")

    @functools.partial(
        pl.kernel,
        mesh=mesh,
        compiler_params=pltpu.CompilerParams(use_tc_tiling_on_sc=False),
        out_type=jax.ShapeDtypeStruct((NW, nch, CH, D), jnp.float32),
        scratch_types=[
            pltpu.VMEM((nch, CH), jnp.int32),
            pltpu.VMEM((NBUF, CH, D), jnp.float32),
            pltpu.SemaphoreType.DMA((NBUF,)),
        ],
    )
    def gather(idx_hbm, table_hbm, out_hbm, idx_v, rows_v, sem):
        wid = lax.axis_index("s") * NC + lax.axis_index("c")
        pltpu.sync_copy(idx_hbm.at[wid], idx_v)

        for b in range(NBUF):
            pltpu.async_copy(table_hbm.at[idx_v.at[b]], rows_v.at[b], sem.at[b])

        def group(g, carry):
            for b in range(NBUF):
                c = g * NBUF + b
                pltpu.make_async_copy(
                    table_hbm.at[idx_v.at[c]], rows_v.at[b], sem.at[b]
                ).wait()
                pltpu.sync_copy(rows_v.at[b], out_hbm.at[wid, c])
                pltpu.async_copy(
                    table_hbm.at[idx_v.at[c + NBUF]], rows_v.at[b], sem.at[b]
                )
            return carry

        lax.fori_loop(0, ngrp - 1, group, 0)

        for b in range(NBUF):
            c = (ngrp - 1) * NBUF + b
            pltpu.make_async_copy(
                table_hbm.at[idx_v.at[c]], rows_v.at[b], sem.at[b]
            ).wait()
            pltpu.sync_copy(rows_v.at[b], out_hbm.at[wid, c])

    return gather


def kernel(indices, weight):
    batch, hist = indices.shape
    total = batch * hist
    idx = indices.reshape(NW, total // (NW * CH), CH).astype(jnp.int32)
    out = _make_gather(total)(idx, weight)
    return out.reshape(batch, hist, D)
